# Initial kernel scaffold; baseline (speedup 1.0000x reference)
import jax
import jax.numpy as jnp
from jax.experimental import pallas as pl


def kernel(pos_u, pos_v, neg_v, U_weight, V_weight):
    raise NotImplementedError("write your pallas kernel here")


# speedup vs baseline: 1.6989x; 1.6989x over previous
import functools

import jax
import jax.numpy as jnp
from jax import lax
from jax.experimental import pallas as pl
from jax.experimental.pallas import tpu as pltpu
from jax.experimental.pallas import tpu_sc as plsc

VOCAB = 1_000_000
DIM = 64
B = 16384
C = 20
K = 5

NC = 2
NS = 16
NW = NC * NS
BW = B // NW
CH = 32
NCHUNK = BW // CH
UC = CH * C
NEGC = CH * K
U_SPLIT = 128
NEG_SPLIT = 80


def _sc_gather_pool(pos_u2d, pos_v, neg_v2d, U_weight, V_weight):
  mesh = plsc.VectorSubcoreMesh(core_axis_name="c", subcore_axis_name="s")

  @functools.partial(
      pl.kernel,
      mesh=mesh,
      compiler_params=pltpu.CompilerParams(use_tc_tiling_on_sc=False),
      out_type=[
          jax.ShapeDtypeStruct((B, DIM), jnp.float32),
          jax.ShapeDtypeStruct((B, DIM), jnp.float32),
          jax.ShapeDtypeStruct((B * K, DIM), jnp.float32),
      ],
      scratch_types=[
          pltpu.VMEM((UC,), jnp.int32),
          pltpu.VMEM((NEGC,), jnp.int32),
          pltpu.VMEM((CH,), jnp.int32),
          pltpu.VMEM((UC, DIM), jnp.float32),
          pltpu.VMEM((NEGC, DIM), jnp.float32),
          pltpu.VMEM((CH, DIM), jnp.float32),
          pltpu.VMEM((CH, DIM), jnp.float32),
          pltpu.SemaphoreType.DMA,
      ],
  )
  def k(pos_u_hbm, pos_v_hbm, neg_v_hbm, u_hbm, v_hbm,
        emb_u_hbm, emb_v_hbm, emb_neg_hbm,
        uidx, nidx, vidx, urows, nrows, vrows, uacc, sem):
    wid = lax.axis_index("s") * NC + lax.axis_index("c")
    base = wid * BW

    def chunk_body(t, carry):
      b0 = base + t * CH
      pltpu.sync_copy(pos_u_hbm.at[pl.ds(b0 * C, UC)], uidx)
      pltpu.sync_copy(neg_v_hbm.at[pl.ds(b0 * K, NEGC)], nidx)
      pltpu.sync_copy(pos_v_hbm.at[pl.ds(b0, CH)], vidx)
      copies = []
      for j in range(UC // U_SPLIT):
        copies.append(pltpu.async_copy(
            u_hbm.at[uidx.at[pl.ds(j * U_SPLIT, U_SPLIT)]],
            urows.at[pl.ds(j * U_SPLIT, U_SPLIT), :], sem))
      for j in range(NEGC // NEG_SPLIT):
        copies.append(pltpu.async_copy(
            v_hbm.at[nidx.at[pl.ds(j * NEG_SPLIT, NEG_SPLIT)]],
            nrows.at[pl.ds(j * NEG_SPLIT, NEG_SPLIT), :], sem))
      copies.append(pltpu.async_copy(v_hbm.at[vidx], vrows, sem))
      for cp in copies:
        cp.wait()

      def item_body(i, c2):
        def c_body(c, accs):
          r = i * C + c
          return tuple(accs[g] + urows[r, pl.ds(g * 16, 16)]
                       for g in range(DIM // 16))
        accs = lax.fori_loop(
            0, C, c_body,
            tuple(jnp.zeros((16,), jnp.float32) for _ in range(DIM // 16)))
        for g in range(DIM // 16):
          uacc[i, pl.ds(g * 16, 16)] = accs[g] * (1.0 / C)
        return c2

      lax.fori_loop(0, CH, item_body, 0)

      pltpu.sync_copy(uacc, emb_u_hbm.at[pl.ds(b0, CH), :])
      pltpu.sync_copy(vrows, emb_v_hbm.at[pl.ds(b0, CH), :])
      pltpu.sync_copy(nrows, emb_neg_hbm.at[pl.ds(b0 * K, NEGC), :])
      return carry

    lax.fori_loop(0, NCHUNK, chunk_body, 0)

  return k(pos_u2d, pos_v, neg_v2d, U_weight, V_weight)


_TC_BLK = 2048


def _tc_loss(emb_u, emb_v, emb_neg):
  grid = B // _TC_BLK

  def body(u_ref, v_ref, n_ref, out_ref):
    i = pl.program_id(0)
    u = u_ref[...]
    v = v_ref[...]
    n = n_ref[...]
    s = jnp.sum(u * v, axis=1)
    s = jnp.clip(s, -10.0, 10.0)
    pos_term = jnp.log1p(jnp.exp(-s))
    u_rep = jnp.broadcast_to(u[:, None, :], (_TC_BLK, K, DIM))
    u_rep = u_rep.reshape(_TC_BLK * K, DIM)
    ns = jnp.sum(n * u_rep, axis=1)
    ns = jnp.clip(ns, -10.0, 10.0)
    neg_term = jnp.log1p(jnp.exp(ns))
    part = (jnp.sum(pos_term) + jnp.sum(neg_term)) * (1.0 / B)

    @pl.when(i == 0)
    def _():
      out_ref[...] = jnp.zeros((1, 1), jnp.float32)

    out_ref[...] += jnp.full((1, 1), part, jnp.float32)

  out = pl.pallas_call(
      body,
      grid=(grid,),
      in_specs=[
          pl.BlockSpec((_TC_BLK, DIM), lambda i: (i, 0)),
          pl.BlockSpec((_TC_BLK, DIM), lambda i: (i, 0)),
          pl.BlockSpec((_TC_BLK * K, DIM), lambda i: (i, 0)),
      ],
      out_specs=pl.BlockSpec((1, 1), lambda i: (0, 0)),
      out_shape=jax.ShapeDtypeStruct((1, 1), jnp.float32),
  )(emb_u, emb_v, emb_neg)
  return out[0, 0]


def kernel(pos_u, pos_v, neg_v, U_weight, V_weight):
  pos_u2d = pos_u.astype(jnp.int32).reshape(B * C)
  neg_v2d = neg_v.astype(jnp.int32).reshape(B * K)
  pos_v1d = pos_v.astype(jnp.int32).reshape(B)
  emb_u, emb_v, emb_neg = _sc_gather_pool(
      pos_u2d, pos_v1d, neg_v2d, U_weight, V_weight)
  return _tc_loss(emb_u, emb_v, emb_neg)
